# baseline (device time: 49882 ns/iter reference)
import jax
import jax.numpy as jnp
from jax import lax
from jax.experimental import pallas as pl
from jax.experimental.pallas import tpu as pltpu

N_DEV = 16
G = 4
CPG = N_DEV // G


def kernel(x, w_mat, scale_x, scale_w):
    m_total, k_loc = x.shape
    k_total, n_out = w_mat.shape
    m_per = m_total // N_DEV
    k_sup = k_total // G

    my_pos = lax.axis_index("i").astype(jnp.int32)
    my_q = my_pos // CPG
    order = jnp.array([0, 3, 1, 2], dtype=jnp.int32)
    qtab = jnp.remainder(my_q + order, G)

    def body(qtab_ref, x_ref, w_ref, sx_ref, sw_ref, out_ref,
             comm_ref, xcat_ref, acc_ref, send_sems, recv_sems):
        s = pl.program_id(0)
        my = lax.axis_index("i").astype(jnp.int32)
        dims = (((1,), (0,)), ((), ()))

        @pl.when(s == 0)
        def _():
            barrier = pltpu.get_barrier_semaphore()
            for d in range(1, N_DEV):
                t = lax.rem(my + d, N_DEV)
                pl.semaphore_signal(barrier, inc=1, device_id=(t,),
                                    device_id_type=pl.DeviceIdType.MESH)
            pl.semaphore_wait(barrier, N_DEV - 1)

            for d in range(1, N_DEV):
                t = lax.rem(my + d, N_DEV)
                pltpu.make_async_remote_copy(
                    src_ref=x_ref.at[pl.ds(t * m_per, m_per), :],
                    dst_ref=comm_ref.at[d],
                    send_sem=send_sems.at[d],
                    recv_sem=recv_sems.at[d],
                    device_id=(t,),
                    device_id_type=pl.DeviceIdType.MESH,
                ).start()

        q = qtab_ref[s]
        for i in range(CPG):
            c = q * CPG + i
            d = lax.rem(my - c + N_DEV, N_DEV)

            @pl.when(c == my)
            def _():
                xcat_ref[:, pl.ds(i * k_loc, k_loc)] = (
                    x_ref[pl.ds(my * m_per, m_per), :].astype(jnp.bfloat16))

            @pl.when(c != my)
            def _():
                rdma = pltpu.make_async_remote_copy(
                    src_ref=x_ref.at[pl.ds(0, m_per), :],
                    dst_ref=comm_ref.at[d],
                    send_sem=send_sems.at[d],
                    recv_sem=recv_sems.at[d],
                    device_id=(my,),
                    device_id_type=pl.DeviceIdType.MESH,
                )
                rdma.wait_recv()
                xcat_ref[:, pl.ds(i * k_loc, k_loc)] = (
                    comm_ref[d].astype(jnp.bfloat16))

        wb = w_ref[...].astype(jnp.bfloat16)

        @pl.when(s == 0)
        def _():
            acc_ref[...] = lax.dot_general(
                xcat_ref[...], wb, dims, preferred_element_type=jnp.float32)

        @pl.when(s > 0)
        def _():
            acc_ref[...] += lax.dot_general(
                xcat_ref[...], wb, dims, preferred_element_type=jnp.float32)

        @pl.when(s == G - 1)
        def _():
            for d in range(1, N_DEV):
                pltpu.make_async_remote_copy(
                    src_ref=x_ref.at[pl.ds(0, m_per), :],
                    dst_ref=comm_ref.at[d],
                    send_sem=send_sems.at[d],
                    recv_sem=recv_sems.at[d],
                    device_id=(my,),
                    device_id_type=pl.DeviceIdType.MESH,
                ).wait_send()
            alpha = sx_ref[0] * sw_ref[0]
            out_ref[...] = jnp.maximum(acc_ref[...] * alpha, 0.0)

    grid_spec = pltpu.PrefetchScalarGridSpec(
        num_scalar_prefetch=1,
        grid=(G,),
        in_specs=[
            pl.BlockSpec((m_total, k_loc), lambda s, qt: (0, 0)),
            pl.BlockSpec((k_sup, n_out), lambda s, qt: (qt[s], 0)),
            pl.BlockSpec(memory_space=pltpu.SMEM),
            pl.BlockSpec(memory_space=pltpu.SMEM),
        ],
        out_specs=pl.BlockSpec((m_per, n_out), lambda s, qt: (0, 0)),
        scratch_shapes=[
            pltpu.VMEM((N_DEV, m_per, k_loc), jnp.int8),
            pltpu.VMEM((m_per, k_sup), jnp.bfloat16),
            pltpu.VMEM((m_per, n_out), jnp.float32),
            pltpu.SemaphoreType.DMA((N_DEV,)),
            pltpu.SemaphoreType.DMA((N_DEV,)),
        ],
    )

    return pl.pallas_call(
        body,
        grid_spec=grid_spec,
        out_shape=jax.ShapeDtypeStruct((m_per, n_out), jnp.float32),
        compiler_params=pltpu.CompilerParams(
            collective_id=0,
            dimension_semantics=("arbitrary",),
            vmem_limit_bytes=100 * 1024 * 1024,
        ),
    )(qtab, x, w_mat, scale_x, scale_w)


# device time: 26909 ns/iter; 1.8537x vs baseline; 1.8537x over previous
import jax
import jax.numpy as jnp
from jax import lax
from jax.experimental import pallas as pl
from jax.experimental.pallas import tpu as pltpu

N_DEV = 16
G = 4
CPG = N_DEV // G


def kernel(x, w_mat, scale_x, scale_w):
    m_total, k_loc = x.shape
    k_total, n_out = w_mat.shape
    m_per = m_total // N_DEV
    k_sup = k_total // G

    my_pos = lax.axis_index("i").astype(jnp.int32)
    my_q = my_pos // CPG
    order = jnp.array([0, 3, 1, 2], dtype=jnp.int32)
    qtab = jnp.remainder(my_q + order, G)

    def body(qtab_ref, x_ref, w_ref, sx_ref, sw_ref, out_ref,
             comm_ref, xcat_ref, acc_ref, send_sems, recv_sems):
        s = pl.program_id(0)
        my = lax.axis_index("i").astype(jnp.int32)
        dims = (((1,), (0,)), ((), ()))

        @pl.when(s == 0)
        def _():
            barrier = pltpu.get_barrier_semaphore()
            for d in range(1, N_DEV):
                t = lax.rem(my + d, N_DEV)
                pl.semaphore_signal(barrier, inc=1, device_id=(t,),
                                    device_id_type=pl.DeviceIdType.MESH)
            pl.semaphore_wait(barrier, N_DEV - 1)

            qbase = (my // CPG) * CPG
            jseq = (list(range(CPG)) +
                    [CPG + j for j in range(CPG)] +
                    [3 * CPG + j for j in range(CPG)] +
                    [2 * CPG + j for j in range(CPG)])
            for j in jseq:
                t = lax.rem(qbase + j, N_DEV)
                d = lax.rem(t - my + N_DEV, N_DEV)

                @pl.when(t != my)
                def _():
                    pltpu.make_async_remote_copy(
                        src_ref=x_ref.at[pl.ds(t * m_per, m_per), :],
                        dst_ref=comm_ref.at[d],
                        send_sem=send_sems.at[d],
                        recv_sem=recv_sems.at[d],
                        device_id=(t,),
                        device_id_type=pl.DeviceIdType.MESH,
                    ).start()

        q = qtab_ref[s]
        for i in range(CPG):
            c = q * CPG + i
            d = lax.rem(my - c + N_DEV, N_DEV)

            @pl.when(c == my)
            def _():
                xcat_ref[:, pl.ds(i * k_loc, k_loc)] = (
                    x_ref[pl.ds(my * m_per, m_per), :].astype(jnp.bfloat16))

            @pl.when(c != my)
            def _():
                rdma = pltpu.make_async_remote_copy(
                    src_ref=x_ref.at[pl.ds(0, m_per), :],
                    dst_ref=comm_ref.at[d],
                    send_sem=send_sems.at[d],
                    recv_sem=recv_sems.at[d],
                    device_id=(my,),
                    device_id_type=pl.DeviceIdType.MESH,
                )
                rdma.wait_recv()
                xcat_ref[:, pl.ds(i * k_loc, k_loc)] = (
                    comm_ref[d].astype(jnp.bfloat16))

        wb = w_ref[...].astype(jnp.bfloat16)

        @pl.when(s == 0)
        def _():
            acc_ref[...] = lax.dot_general(
                xcat_ref[...], wb, dims, preferred_element_type=jnp.float32)

        @pl.when(s > 0)
        def _():
            acc_ref[...] += lax.dot_general(
                xcat_ref[...], wb, dims, preferred_element_type=jnp.float32)

        @pl.when(s == G - 1)
        def _():
            for d in range(1, N_DEV):
                pltpu.make_async_remote_copy(
                    src_ref=x_ref.at[pl.ds(0, m_per), :],
                    dst_ref=comm_ref.at[d],
                    send_sem=send_sems.at[d],
                    recv_sem=recv_sems.at[d],
                    device_id=(my,),
                    device_id_type=pl.DeviceIdType.MESH,
                ).wait_send()
            alpha = sx_ref[0] * sw_ref[0]
            out_ref[...] = jnp.maximum(acc_ref[...] * alpha, 0.0)

    grid_spec = pltpu.PrefetchScalarGridSpec(
        num_scalar_prefetch=1,
        grid=(G,),
        in_specs=[
            pl.BlockSpec((m_total, k_loc), lambda s, qt: (0, 0)),
            pl.BlockSpec((k_sup, n_out), lambda s, qt: (qt[s], 0)),
            pl.BlockSpec(memory_space=pltpu.SMEM),
            pl.BlockSpec(memory_space=pltpu.SMEM),
        ],
        out_specs=pl.BlockSpec((m_per, n_out), lambda s, qt: (0, 0)),
        scratch_shapes=[
            pltpu.VMEM((N_DEV, m_per, k_loc), jnp.int8),
            pltpu.VMEM((m_per, k_sup), jnp.bfloat16),
            pltpu.VMEM((m_per, n_out), jnp.float32),
            pltpu.SemaphoreType.DMA((N_DEV,)),
            pltpu.SemaphoreType.DMA((N_DEV,)),
        ],
    )

    return pl.pallas_call(
        body,
        grid_spec=grid_spec,
        out_shape=jax.ShapeDtypeStruct((m_per, n_out), jnp.float32),
        compiler_params=pltpu.CompilerParams(
            collective_id=0,
            dimension_semantics=("arbitrary",),
            vmem_limit_bytes=100 * 1024 * 1024,
        ),
    )(qtab, x, w_mat, scale_x, scale_w)
